# baseline (speedup 1.0000x reference)
import functools

import jax
import jax.numpy as jnp
from jax.experimental import pallas as pl
from jax.experimental.pallas import tpu as pltpu

_NEG_SLOPE = 0.2
_EPS = 1e-8


def _pixnorm_lrelu_halves(gs):
    ssq = gs[0] * gs[0]
    for g in gs[1:]:
        ssq = ssq + g * g
    norm = jnp.sqrt(jnp.sum(ssq, axis=1, keepdims=True))
    r = pl.reciprocal(norm + _EPS, approx=True)
    out = []
    for g in gs:
        t = g * r
        out.append(jnp.maximum(t, _NEG_SLOPE * t))
    return out


def _fused_kernel(x_ref, w1_ref, b1_ref, w2_ref, b2_ref, wr0_ref, wr1_ref,
                  br_ref, o_ref, *, H, W, E):
    for e in range(E):
        _one_element(x_ref, w1_ref, b1_ref, w2_ref, b2_ref, wr0_ref, wr1_ref,
                     br_ref, o_ref, e, H, W)


def _one_element(x_ref, w1_ref, b1_ref, w2_ref, b2_ref, wr0_ref, wr1_ref,
                 br_ref, o_ref, e, H, W):
    S = H * W
    x = x_ref[e].astype(jnp.bfloat16)
    bf = jnp.bfloat16

    j = jax.lax.broadcasted_iota(jnp.int32, (S, 1), 0) % W
    m_hi = (j < (W - 1))
    m_lo = (j > 0)

    def colshift(v, sx):
        if sx == 0:
            return v
        z = jnp.zeros((1, v.shape[1]), v.dtype)
        if sx == 1:
            return jnp.concatenate([v[1:], z], axis=0) * m_hi.astype(v.dtype)
        return jnp.concatenate([z, v[:-1]], axis=0) * m_lo.astype(v.dtype)

    def rowshift(v, sy):
        if sy == 0:
            return v
        z = jnp.zeros((W, v.shape[1]), v.dtype)
        if sy == 1:
            return jnp.concatenate([v[W:], z], axis=0)
        return jnp.concatenate([z, v[:S - W]], axis=0)

    def shift2(v, sy, sx):
        return rowshift(colshift(v, sx), sy)

    tnums = (((0,), (0,)), ((), ()))
    C = x.shape[0]
    Co = w2_ref.shape[2]
    NH = 2 if (C % 2 == 0 and Co % 2 == 0) else 1
    C2, Co2 = C // NH, Co // NH

    h1 = {}
    for hh in range(NH):
        cs = slice(hh * C2, (hh + 1) * C2)
        t1 = [jax.lax.dot_general(x, w1_ref[o][:, cs], tnums,
                                  preferred_element_type=jnp.float32).astype(bf)
              for o in range(9)]
        yc = {(0, 0): t1[0:3],
              (0, 1): [t1[3 + k] + t1[6 + k] for k in range(3)],
              (1, 0): [t1[0 + k] + t1[3 + k] for k in range(3)],
              (1, 1): t1[6:9]}
        for p in (0, 1):
            for q in (0, 1):
                acc = None
                for a in (0, 1):
                    r = yc[(p, a)]
                    for b in (0, 1):
                        if (q, b) == (0, 0):
                            u = r[0]
                        elif (q, b) == (0, 1):
                            u = r[1] + r[2]
                        elif (q, b) == (1, 0):
                            u = r[0] + r[1]
                        else:
                            u = r[2]
                        s = shift2(u, a - 1 + p, b - 1 + q)
                        acc = s if acc is None else acc + s
                h1.setdefault((p, q), []).append(
                    acc.astype(jnp.float32) + b1_ref[...][:, cs])
    for pq, gs in h1.items():
        h1[pq] = jnp.concatenate(
            [g.astype(bf) for g in _pixnorm_lrelu_halves(gs)], axis=1)

    WIN = ((1, -1), (0, 0), (1, 0), (0, 1))
    BT = ((1, 0, -1, 0), (0, 1, 1, 0), (0, -1, 1, 0), (0, 1, 0, -1))
    AT = ((1, 1, 1, 0), (0, 1, -1, -1))
    ccache = {}

    def dtile(u, v):
        py, sy = WIN[u]
        px, sx = WIN[v]
        key = (py, px, sx)
        if key not in ccache:
            ccache[key] = colshift(h1[(py, px)], sx)
        return rowshift(ccache[key], sy)

    def combo(coeffs, getter):
        acc = None
        for i, c in enumerate(coeffs):
            if c == 0:
                continue
            t = getter(i) if c == 1 else -getter(i)
            acc = t if acc is None else acc + t
        return acc

    Vy = [[combo(BT[k], lambda u: dtile(u, v)) for v in range(4)]
          for k in range(4)]
    Yacc = {}
    for k in range(4):
        for l in range(4):
            V = combo(BT[l], lambda v: Vy[k][v])
            M = jnp.dot(V, w2_ref[k * 4 + l],
                        preferred_element_type=jnp.float32)
            for p in (0, 1):
                if AT[p][k] == 0:
                    continue
                for q in (0, 1):
                    c = AT[p][k] * AT[q][l]
                    if c == 0:
                        continue
                    t = M if c == 1 else -M
                    Yacc[(p, q)] = t if (p, q) not in Yacc \
                        else Yacc[(p, q)] + t
    h2 = {}
    for pq, y in Yacc.items():
        h2[pq] = _pixnorm_lrelu_halves([y + b2_ref[...]])[0].astype(bf)

    base = jax.lax.dot_general(x, wr0_ref[...], tnums,
                               preferred_element_type=jnp.float32) \
        + br_ref[...]
    h2all = jnp.concatenate(
        [h2[pq] for pq in ((0, 0), (0, 1), (1, 0), (1, 1))], axis=0)
    o_ref[e] = jnp.dot(h2all, wr1_ref[...],
                       preferred_element_type=jnp.float32) \
        + jnp.concatenate([base] * 4, axis=0)


def _const_spec(a):
    return pl.BlockSpec(a.shape, lambda n: (0,) * a.ndim)


def kernel(x, conv1_w, conv1_b, conv2_w, conv2_b,
           rgb0_w, rgb0_b, rgb1_w, rgb1_b, alpha):
    N, C, H, W = x.shape
    Co = conv2_w.shape[3]
    S = H * W
    bf = jnp.bfloat16

    xt = x.reshape(N, C, S)

    w1e = conv1_w.reshape(9, C, C).astype(bf)
    G = jnp.asarray([[1.0, 0.0, 0.0], [0.5, 0.5, 0.5],
                     [0.5, -0.5, 0.5], [0.0, 0.0, 1.0]], jnp.float32)
    w2 = jnp.einsum('ku,lv,uvio->klio', G, G,
                    conv2_w.astype(jnp.float32)).reshape(16, C, Co).astype(bf)
    b1 = conv1_b.reshape(1, C).astype(jnp.float32)
    b2 = conv2_b.reshape(1, Co).astype(jnp.float32)

    a = jnp.asarray(alpha, jnp.float32)
    wr0 = ((1.0 - a) * rgb0_w).astype(bf)
    wr1 = (a * rgb1_w).astype(bf)
    br = ((1.0 - a) * rgb0_b + a * rgb1_b).reshape(1, 3).astype(jnp.float32)

    E = 8 if N % 8 == 0 else (2 if N % 2 == 0 else 1)
    out = pl.pallas_call(
        functools.partial(_fused_kernel, H=H, W=W, E=E),
        out_shape=jax.ShapeDtypeStruct((N, 4 * S, 3), jnp.float32),
        grid=(N // E,),
        in_specs=[
            pl.BlockSpec((E, C, S), lambda n: (n, 0, 0)),
            _const_spec(w1e), _const_spec(b1),
            _const_spec(w2), _const_spec(b2),
            _const_spec(wr0), _const_spec(wr1), _const_spec(br),
        ],
        out_specs=pl.BlockSpec((E, 4 * S, 3), lambda n: (n, 0, 0)),
        compiler_params=pltpu.CompilerParams(
            dimension_semantics=("parallel",)),
    )(xt, w1e, b1, w2, b2, wr0, wr1, br)

    o = out.reshape(N, 2, 2, H, W, 3)
    return o.transpose(0, 5, 3, 1, 4, 2).reshape(N, 3, 2 * H, 2 * W)


# speedup vs baseline: 1.2057x; 1.2057x over previous
import functools

import jax
import jax.numpy as jnp
from jax.experimental import pallas as pl
from jax.experimental.pallas import tpu as pltpu

_NEG_SLOPE = 0.2
_EPS = 1e-8


def _pixnorm_lrelu_halves(gs):
    ssq = gs[0] * gs[0]
    for g in gs[1:]:
        ssq = ssq + g * g
    norm = jnp.sqrt(jnp.sum(ssq, axis=1, keepdims=True))
    r = pl.reciprocal(norm + _EPS, approx=True)
    out = []
    for g in gs:
        t = g * r
        out.append(jnp.maximum(t, _NEG_SLOPE * t))
    return out


def _fused_kernel(x_ref, w1_ref, b1_ref, w2_ref, b2_ref, wr0_ref, wr1_ref,
                  br_ref, o_ref, *, H, W, E):
    for e in range(E):
        _one_element(x_ref, w1_ref, b1_ref, w2_ref, b2_ref, wr0_ref, wr1_ref,
                     br_ref, o_ref, e, H, W)


def _one_element(x_ref, w1_ref, b1_ref, w2_ref, b2_ref, wr0_ref, wr1_ref,
                 br_ref, o_ref, e, H, W):
    S = H * W
    x = x_ref[e].astype(jnp.bfloat16)
    bf = jnp.bfloat16

    j = jax.lax.broadcasted_iota(jnp.int32, (S, 1), 0) % W
    m_hi = (j < (W - 1))
    m_lo = (j > 0)

    def colshift(v, sx):
        if sx == 0:
            return v
        z = jnp.zeros((1, v.shape[1]), v.dtype)
        if sx == 1:
            return jnp.concatenate([v[1:], z], axis=0) * m_hi.astype(v.dtype)
        return jnp.concatenate([z, v[:-1]], axis=0) * m_lo.astype(v.dtype)

    def rowshift(v, sy):
        if sy == 0:
            return v
        z = jnp.zeros((W, v.shape[1]), v.dtype)
        if sy == 1:
            return jnp.concatenate([v[W:], z], axis=0)
        return jnp.concatenate([z, v[:S - W]], axis=0)

    def shift2(v, sy, sx):
        return rowshift(colshift(v, sx), sy)

    tnums = (((0,), (0,)), ((), ()))
    C = x.shape[0]
    Co = w2_ref.shape[2]
    NH = 2 if (C % 2 == 0 and Co % 2 == 0) else 1
    C2, Co2 = C // NH, Co // NH

    h1 = {}
    for hh in range(NH):
        cs = slice(hh * C2, (hh + 1) * C2)
        t1 = [jax.lax.dot_general(x, w1_ref[o][:, cs], tnums,
                                  preferred_element_type=jnp.float32).astype(bf)
              for o in range(9)]
        yc = {(0, 0): t1[0:3],
              (0, 1): [t1[3 + k] + t1[6 + k] for k in range(3)],
              (1, 0): [t1[0 + k] + t1[3 + k] for k in range(3)],
              (1, 1): t1[6:9]}
        for p in (0, 1):
            for q in (0, 1):
                acc = None
                for a in (0, 1):
                    r = yc[(p, a)]
                    for b in (0, 1):
                        if (q, b) == (0, 0):
                            u = r[0]
                        elif (q, b) == (0, 1):
                            u = r[1] + r[2]
                        elif (q, b) == (1, 0):
                            u = r[0] + r[1]
                        else:
                            u = r[2]
                        s = shift2(u, a - 1 + p, b - 1 + q)
                        acc = s if acc is None else acc + s
                h1.setdefault((p, q), []).append(
                    acc.astype(jnp.float32) + b1_ref[...][:, cs])
    for pq, gs in h1.items():
        h1[pq] = jnp.concatenate(
            [g.astype(bf) for g in _pixnorm_lrelu_halves(gs)], axis=1)

    WIN = ((1, -1), (0, 0), (1, 0), (0, 1))
    BT = ((1, 0, -1, 0), (0, 1, 1, 0), (0, -1, 1, 0), (0, 1, 0, -1))
    AT = ((1, 1, 1, 0), (0, 1, -1, -1))
    ccache = {}

    def dtile(u, v):
        py, sy = WIN[u]
        px, sx = WIN[v]
        key = (py, px, sx)
        if key not in ccache:
            ccache[key] = colshift(h1[(py, px)], sx)
        return rowshift(ccache[key], sy)

    def combo(coeffs, getter):
        acc = None
        for i, c in enumerate(coeffs):
            if c == 0:
                continue
            t = getter(i) if c == 1 else -getter(i)
            acc = t if acc is None else acc + t
        return acc

    Vy = [[combo(BT[k], lambda u: dtile(u, v)) for v in range(4)]
          for k in range(4)]
    Yacc = {}
    for k in range(4):
        for l in range(4):
            V = combo(BT[l], lambda v: Vy[k][v])
            M = jnp.dot(V, w2_ref[k * 4 + l],
                        preferred_element_type=jnp.float32)
            for p in (0, 1):
                if AT[p][k] == 0:
                    continue
                for q in (0, 1):
                    c = AT[p][k] * AT[q][l]
                    if c == 0:
                        continue
                    t = M if c == 1 else -M
                    Yacc[(p, q)] = t if (p, q) not in Yacc \
                        else Yacc[(p, q)] + t
    h2 = {}
    for pq, y in Yacc.items():
        h2[pq] = _pixnorm_lrelu_halves([y + b2_ref[...]])[0].astype(bf)

    base = jax.lax.dot_general(x, wr0_ref[...], tnums,
                               preferred_element_type=jnp.float32) \
        + br_ref[...]
    h2all = jnp.concatenate(
        [h2[pq] for pq in ((0, 0), (0, 1), (1, 0), (1, 1))], axis=0)
    o_ref[e] = jnp.dot(h2all, wr1_ref[...],
                       preferred_element_type=jnp.float32) \
        + jnp.concatenate([base] * 4, axis=0)


def _const_spec(a):
    return pl.BlockSpec(a.shape, lambda n: (0,) * a.ndim)


def kernel(x, conv1_w, conv1_b, conv2_w, conv2_b,
           rgb0_w, rgb0_b, rgb1_w, rgb1_b, alpha):
    N, C, H, W = x.shape
    Co = conv2_w.shape[3]
    S = H * W
    bf = jnp.bfloat16

    xt = x.reshape(N, C, S)

    w1e = conv1_w.reshape(9, C, C).astype(bf)
    G = jnp.asarray([[1.0, 0.0, 0.0], [0.5, 0.5, 0.5],
                     [0.5, -0.5, 0.5], [0.0, 0.0, 1.0]], jnp.float32)
    w2 = jnp.einsum('ku,lv,uvio->klio', G, G,
                    conv2_w.astype(jnp.float32)).reshape(16, C, Co).astype(bf)
    b1 = conv1_b.reshape(1, C).astype(jnp.float32)
    b2 = conv2_b.reshape(1, Co).astype(jnp.float32)

    a = jnp.asarray(alpha, jnp.float32)
    wr0 = ((1.0 - a) * rgb0_w).astype(bf)
    wr1 = (a * rgb1_w).astype(bf)
    br = ((1.0 - a) * rgb0_b + a * rgb1_b).reshape(1, 3).astype(jnp.float32)

    E = 4 if N % 4 == 0 else (2 if N % 2 == 0 else 1)
    out = pl.pallas_call(
        functools.partial(_fused_kernel, H=H, W=W, E=E),
        out_shape=jax.ShapeDtypeStruct((N, 4 * S, 3), jnp.float32),
        grid=(N // E,),
        in_specs=[
            pl.BlockSpec((E, C, S), lambda n: (n, 0, 0)),
            _const_spec(w1e), _const_spec(b1),
            _const_spec(w2), _const_spec(b2),
            _const_spec(wr0), _const_spec(wr1), _const_spec(br),
        ],
        out_specs=pl.BlockSpec((E, 4 * S, 3), lambda n: (n, 0, 0)),
        compiler_params=pltpu.CompilerParams(
            dimension_semantics=("parallel",)),
    )(xt, w1e, b1, w2, b2, wr0, wr1, br)

    o = out.reshape(N, 2, 2, H, W, 3)
    return o.transpose(0, 5, 3, 1, 4, 2).reshape(N, 3, 2 * H, 2 * W)
